# baseline (speedup 1.0000x reference)
import functools

import jax
import jax.numpy as jnp
from jax import lax
from jax.experimental import pallas as pl
from jax.experimental.pallas import tpu as pltpu
from jax.experimental.pallas import tpu_sc as plsc

NC = 2
NS = 16
NW = NC * NS

FEAT = 256
NLANE = 16
K = 10
G = K + 1
CHN = 8
NSLOT = 4

NCH0 = 56
NCH1 = 24
B_PAD = NS * (NCH0 + NCH1) * CHN


def _sc_gather_fn():
    core0_cols = NS * NCH0 * CHN
    mesh = plsc.VectorSubcoreMesh(core_axis_name="c", subcore_axis_name="s")

    @functools.partial(
        pl.kernel,
        mesh=mesh,
        out_type=(
            jax.ShapeDtypeStruct((B_PAD, FEAT), jnp.float32),
            jax.ShapeDtypeStruct((B_PAD, FEAT), jnp.float32),
        ),
        scratch_types=(
            pltpu.VMEM((NCH0 * CHN * G,), jnp.int32),
            *[pltpu.VMEM((CHN * G, FEAT), jnp.float32)
              for _ in range(NSLOT)],
            *[pltpu.VMEM((CHN, FEAT), jnp.float32)
              for _ in range(NSLOT)],
            *[pltpu.VMEM((CHN, FEAT), jnp.float32)
              for _ in range(NSLOT)],
            *[pltpu.SemaphoreType.DMA for _ in range(NSLOT)],
            *[pltpu.SemaphoreType.DMA for _ in range(NSLOT)],
        ),
    )
    def sc_gather(feat_hbm, idx_hbm, self_out, neigh_out, idx_v, *slot_refs):
        bufs = slot_refs[:NSLOT]
        sstages = slot_refs[NSLOT:2 * NSLOT]
        nstages = slot_refs[2 * NSLOT:3 * NSLOT]
        sems_i = slot_refs[3 * NSLOT:4 * NSLOT]
        sems_o = slot_refs[4 * NSLOT:5 * NSLOT]
        cid = lax.axis_index("c")
        sid = lax.axis_index("s")
        n_chunks = jnp.where(cid == 0, NCH0, NCH1)
        base = jnp.where(cid == 0, sid * (NCH0 * CHN),
                         core0_cols + sid * (NCH1 * CHN))

        @pl.when(cid == 0)
        def _stage0():
            pltpu.sync_copy(idx_hbm.at[pl.ds(base * G, NCH0 * CHN * G)], idx_v)

        @pl.when(cid != 0)
        def _stage1():
            pltpu.sync_copy(idx_hbm.at[pl.ds(base * G, NCH1 * CHN * G)],
                            idx_v.at[pl.ds(0, NCH1 * CHN * G)])

        def in_copies(ic, buf, sem):
            return [
                pltpu.make_async_copy(
                    feat_hbm.at[idx_v.at[pl.ds(ic * (CHN * G), CHN * G)]],
                    buf, sem)
            ]

        def out_copies(ic, sstage, nstage, sem):
            dst = pl.ds(base + ic * CHN, CHN)
            return [
                pltpu.make_async_copy(sstage, self_out.at[dst], sem),
                pltpu.make_async_copy(nstage, neigh_out.at[dst], sem),
            ]

        def reduce_chunk(buf, sstage, nstage):
            @pl.loop(0, CHN)
            def _col(c):
                rbase = c * G
                for d in range(FEAT // NLANE):
                    sl = pl.ds(d * NLANE, NLANE)
                    sstage[c, sl] = buf[rbase, sl]
                    acc = buf[rbase + 1, sl]
                    for j in range(2, G):
                        acc = acc + buf[rbase + j, sl]
                    nstage[c, sl] = acc * jnp.float32(1.0 / K)

        slots = tuple(zip(bufs, sstages, nstages, sems_i, sems_o))

        for b, (buf, _, _, sem_i, _) in enumerate(slots):
            for c in in_copies(b, buf, sem_i):
                c.start()

        @pl.loop(0, n_chunks, step=NSLOT)
        def _chunk(i):
            for b, (buf, sstage, nstage, sem_i, sem_o) in enumerate(slots):
                ic = i + b
                for c in in_copies(ic, buf, sem_i):
                    c.wait()

                @pl.when(ic >= NSLOT)
                def _drain():
                    for c in out_copies(ic - NSLOT, sstage, nstage, sem_o):
                        c.wait()

                reduce_chunk(buf, sstage, nstage)

                @pl.when(ic + NSLOT < n_chunks)
                def _refire():
                    for c in in_copies(ic + NSLOT, buf, sem_i):
                        c.start()

                for c in out_copies(ic, sstage, nstage, sem_o):
                    c.start()

        for b, (buf, sstage, nstage, _, sem_o) in enumerate(slots):
            for c in out_copies(n_chunks - NSLOT + b, sstage, nstage, sem_o):
                c.wait()

    return sc_gather


def _tc_body(w_ref, s_ref, n_ref, o_ref):
    w = w_ref[...].astype(jnp.bfloat16)
    s = s_ref[...].astype(jnp.bfloat16)
    n = n_ref[...].astype(jnp.bfloat16)
    dn = (((1,), (1,)), ((), ()))
    acc = lax.dot_general(w[:, :FEAT], s, dn, preferred_element_type=jnp.float32)
    acc = acc + lax.dot_general(w[:, FEAT:], n, dn,
                                preferred_element_type=jnp.float32)
    o_ref[...] = jnp.maximum(acc, 0.0)


def _tc_matmul(weight, self_f, neigh_m, b, tb):
    grid = (B_PAD // tb,)
    return pl.pallas_call(
        _tc_body,
        grid=grid,
        in_specs=[
            pl.BlockSpec((FEAT, 2 * FEAT), lambda i: (0, 0)),
            pl.BlockSpec((tb, FEAT), lambda i: (i, 0)),
            pl.BlockSpec((tb, FEAT), lambda i: (i, 0)),
        ],
        out_specs=pl.BlockSpec((FEAT, tb), lambda i: (0, i)),
        out_shape=jax.ShapeDtypeStruct((FEAT, b), jnp.float32),
    )(weight, self_f, neigh_m)


def kernel(features, weight, nodes, neigh_idx):
    b = nodes.shape[0]

    nodes_p = jnp.zeros((B_PAD,), jnp.int32).at[:b].set(nodes.astype(jnp.int32))
    neigh_p = jnp.zeros((B_PAD, K), jnp.int32).at[:b].set(
        neigh_idx.astype(jnp.int32))
    idx_flat = jnp.concatenate([nodes_p[:, None], neigh_p], axis=1).reshape(-1)

    self_f, neigh_m = _sc_gather_fn()(features, idx_flat)
    return _tc_matmul(weight, self_f, neigh_m, b, tb=1024)


# speedup vs baseline: 1.0246x; 1.0246x over previous
import functools

import jax
import jax.numpy as jnp
from jax import lax
from jax.experimental import pallas as pl
from jax.experimental.pallas import tpu as pltpu
from jax.experimental.pallas import tpu_sc as plsc

NC = 2
NS = 16
NW = NC * NS

FEAT = 256
NLANE = 16
K = 10
G = K + 1
CHN = 16
NSTREAM = 2
CPS = CHN // NSTREAM

NCH0 = 28
NCH1 = 12
B_PAD = NS * (NCH0 + NCH1) * CHN


def _sc_gather_fn():
    core0_cols = NS * NCH0 * CHN
    mesh = plsc.VectorSubcoreMesh(core_axis_name="c", subcore_axis_name="s")

    @functools.partial(
        pl.kernel,
        mesh=mesh,
        out_type=(
            jax.ShapeDtypeStruct((B_PAD, FEAT), jnp.float32),
            jax.ShapeDtypeStruct((B_PAD, FEAT), jnp.float32),
        ),
        scratch_types=(
            pltpu.VMEM((NCH0 * CHN * G,), jnp.int32),
            pltpu.VMEM((CHN * G, FEAT), jnp.float32),
            pltpu.VMEM((CHN * G, FEAT), jnp.float32),
            pltpu.VMEM((CHN, FEAT), jnp.float32),
            pltpu.VMEM((CHN, FEAT), jnp.float32),
            pltpu.VMEM((CHN, FEAT), jnp.float32),
            pltpu.VMEM((CHN, FEAT), jnp.float32),
            pltpu.SemaphoreType.DMA,
            pltpu.SemaphoreType.DMA,
            pltpu.SemaphoreType.DMA,
            pltpu.SemaphoreType.DMA,
        ),
    )
    def sc_gather(feat_hbm, idx_hbm, self_out, neigh_out,
                  idx_v, buf0, buf1, ss0, ss1, ns0, ns1,
                  sem_i0, sem_i1, sem_o0, sem_o1):
        cid = lax.axis_index("c")
        sid = lax.axis_index("s")
        n_chunks = jnp.where(cid == 0, NCH0, NCH1)
        base = jnp.where(cid == 0, sid * (NCH0 * CHN),
                         core0_cols + sid * (NCH1 * CHN))

        @pl.when(cid == 0)
        def _stage0():
            pltpu.sync_copy(idx_hbm.at[pl.ds(base * G, NCH0 * CHN * G)], idx_v)

        @pl.when(cid != 0)
        def _stage1():
            pltpu.sync_copy(idx_hbm.at[pl.ds(base * G, NCH1 * CHN * G)],
                            idx_v.at[pl.ds(0, NCH1 * CHN * G)])

        def in_copies(ic, buf, sem):
            return [
                pltpu.make_async_copy(
                    feat_hbm.at[idx_v.at[pl.ds((ic * CHN + s * CPS) * G,
                                               CPS * G)]],
                    buf.at[pl.ds(s * CPS * G, CPS * G)],
                    sem)
                for s in range(NSTREAM)
            ]

        def out_copies(ic, sstage, nstage, sem):
            dst = pl.ds(base + ic * CHN, CHN)
            return [
                pltpu.make_async_copy(sstage, self_out.at[dst], sem),
                pltpu.make_async_copy(nstage, neigh_out.at[dst], sem),
            ]

        def reduce_chunk(buf, sstage, nstage):
            @pl.loop(0, CHN)
            def _col(c):
                rbase = c * G
                for d in range(FEAT // NLANE):
                    sl = pl.ds(d * NLANE, NLANE)
                    sstage[c, sl] = buf[rbase, sl]
                    acc = buf[rbase + 1, sl]
                    for j in range(2, G):
                        acc = acc + buf[rbase + j, sl]
                    nstage[c, sl] = acc * jnp.float32(1.0 / K)

        slots = ((buf0, ss0, ns0, sem_i0, sem_o0),
                 (buf1, ss1, ns1, sem_i1, sem_o1))

        for b, (buf, _, _, sem_i, _) in enumerate(slots):
            for c in in_copies(b, buf, sem_i):
                c.start()

        @pl.loop(0, n_chunks, step=2)
        def _chunk(i):
            for b, (buf, sstage, nstage, sem_i, sem_o) in enumerate(slots):
                ic = i + b
                for c in in_copies(ic, buf, sem_i):
                    c.wait()

                @pl.when(ic >= 2)
                def _drain():
                    for c in out_copies(ic - 2, sstage, nstage, sem_o):
                        c.wait()

                reduce_chunk(buf, sstage, nstage)

                @pl.when(ic + 2 < n_chunks)
                def _refire():
                    for c in in_copies(ic + 2, buf, sem_i):
                        c.start()

                for c in out_copies(ic, sstage, nstage, sem_o):
                    c.start()

        for b, (buf, sstage, nstage, _, sem_o) in enumerate(slots):
            for c in out_copies(n_chunks - 2 + b, sstage, nstage, sem_o):
                c.wait()

    return sc_gather


def _tc_body(w_ref, s_ref, n_ref, o_ref):
    w = w_ref[...].astype(jnp.bfloat16)
    s = s_ref[...].astype(jnp.bfloat16)
    n = n_ref[...].astype(jnp.bfloat16)
    dn = (((1,), (1,)), ((), ()))
    acc = lax.dot_general(w[:, :FEAT], s, dn, preferred_element_type=jnp.float32)
    acc = acc + lax.dot_general(w[:, FEAT:], n, dn,
                                preferred_element_type=jnp.float32)
    o_ref[...] = jnp.maximum(acc, 0.0)


def _tc_matmul(weight, self_f, neigh_m, b, tb):
    grid = (B_PAD // tb,)
    return pl.pallas_call(
        _tc_body,
        grid=grid,
        in_specs=[
            pl.BlockSpec((FEAT, 2 * FEAT), lambda i: (0, 0)),
            pl.BlockSpec((tb, FEAT), lambda i: (i, 0)),
            pl.BlockSpec((tb, FEAT), lambda i: (i, 0)),
        ],
        out_specs=pl.BlockSpec((FEAT, tb), lambda i: (0, i)),
        out_shape=jax.ShapeDtypeStruct((FEAT, b), jnp.float32),
    )(weight, self_f, neigh_m)


def kernel(features, weight, nodes, neigh_idx):
    b = nodes.shape[0]

    nodes_p = jnp.zeros((B_PAD,), jnp.int32).at[:b].set(nodes.astype(jnp.int32))
    neigh_p = jnp.zeros((B_PAD, K), jnp.int32).at[:b].set(
        neigh_idx.astype(jnp.int32))
    idx_flat = jnp.concatenate([nodes_p[:, None], neigh_p], axis=1).reshape(-1)

    self_f, neigh_m = _sc_gather_fn()(features, idx_flat)
    return _tc_matmul(weight, self_f, neigh_m, b, tb=1024)
